# Initial kernel scaffold; baseline (speedup 1.0000x reference)
import jax
import jax.numpy as jnp
from jax.experimental import pallas as pl


def kernel(x, W_enc, b_enc, W_gat, att_src, att_dst, b_gat, gn1_w, gn1_b, gn1_ms, W_tag, b_tag, gn2_w, gn2_b, gn2_ms, W1, W2, b_gc, Wc1, bc1, Wc2, bc2, edge_index, batch):
    raise NotImplementedError("write your pallas kernel here")


# speedup vs baseline: 1.0013x; 1.0013x over previous
import functools

import jax
import jax.numpy as jnp
from jax import lax
from jax.experimental import pallas as pl
from jax.experimental.pallas import tpu as pltpu

N = 100000
E = 3200000
G = 64
F = 8
H = 64
HEADS = 4
DH = 16
K = 3


def _gelu_exact(v):
    return 0.5 * v * (1.0 + lax.erf(v * (2.0 ** -0.5)))


def _head_body(hcat_ref, wc1_ref, bc1_ref, wc2_ref, bc2_ref, out_ref):
    h = hcat_ref[...] @ wc1_ref[...] + bc1_ref[...]
    h = _gelu_exact(h)
    logits = h @ wc2_ref[...] + bc2_ref[...]
    out_ref[...] = jax.nn.log_softmax(logits, axis=-1)


def _head(hcat, Wc1, bc1, Wc2, bc2):
    return pl.pallas_call(
        _head_body,
        out_shape=jax.ShapeDtypeStruct((G, 2), jnp.float32),
    )(hcat, Wc1, bc1.reshape(1, H), Wc2, bc2.reshape(1, 2))


def _graph_norm(x, batch, counts, w, b, ms):
    mean = jax.ops.segment_sum(x, batch, G) / counts
    out = x - mean[batch] * ms
    var = jax.ops.segment_sum(out * out, batch, G) / counts
    out = out / jnp.sqrt(var + 1e-5)[batch]
    return out * w + b


def kernel(x, W_enc, b_enc, W_gat, att_src, att_dst, b_gat, gn1_w, gn1_b,
           gn1_ms, W_tag, b_tag, gn2_w, gn2_b, gn2_ms, W1, W2, b_gc, Wc1,
           bc1, Wc2, bc2, edge_index, batch):
    src = edge_index[0]
    dst = edge_index[1]
    x = jax.nn.gelu(x @ W_enc + b_enc, approximate=False)
    loop = jnp.arange(N, dtype=src.dtype)
    s2 = jnp.concatenate([src, loop])
    d2 = jnp.concatenate([dst, loop])
    xw = (x @ W_gat).reshape(N, HEADS, DH)
    a_s = jnp.sum(xw * att_src[None], axis=-1)
    a_d = jnp.sum(xw * att_dst[None], axis=-1)
    e = jax.nn.leaky_relu(a_s[s2] + a_d[d2], 0.2)
    emax = jax.ops.segment_max(e, d2, N)
    emax = jnp.where(jnp.isfinite(emax), emax, 0.0)
    ee = jnp.exp(e - emax[d2])
    denom = jax.ops.segment_sum(ee, d2, N)
    alpha = ee / (denom[d2] + 1e-16)
    gat = jax.ops.segment_sum(xw[s2] * alpha[:, :, None], d2, N).reshape(N, H) + b_gat
    counts = jnp.maximum(jax.ops.segment_sum(jnp.ones((N, 1), jnp.float32), batch, G), 1.0)
    x = jax.nn.relu(_graph_norm(gat, batch, counts, gn1_w, gn1_b, gn1_ms))
    deg = jax.ops.segment_sum(jnp.ones((E,), jnp.float32), dst, N)
    dis = jnp.where(deg > 0, 1.0 / jnp.sqrt(deg), 0.0)
    nrm = dis[src] * dis[dst]
    out = x @ W_tag[0]
    h = x
    for k in range(1, K + 1):
        h = jax.ops.segment_sum(h[src] * nrm[:, None], dst, N)
        out = out + h @ W_tag[k]
    out = out + b_tag
    x = jax.nn.relu(_graph_norm(out, batch, counts, gn2_w, gn2_b, gn2_ms))
    agg = jax.ops.segment_sum(x[src], dst, N)
    x = jax.nn.relu(x @ W1 + agg @ W2 + b_gc)
    xm = jax.ops.segment_max(x, batch, G)
    xm = jnp.where(jnp.isfinite(xm), xm, 0.0)
    xM = jax.ops.segment_sum(x, batch, G) / counts
    hcat = jnp.concatenate([xm, xM], axis=1)
    return _head(hcat, Wc1, bc1, Wc2, bc2)


# speedup vs baseline: 1.2551x; 1.2535x over previous
import functools

import jax
import jax.numpy as jnp
from jax import lax
from jax.experimental import pallas as pl
from jax.experimental.pallas import tpu as pltpu
from jax.experimental.pallas import tpu_sc as plsc

N = 100000
E = 3200000
G = 64
F = 8
H = 64
HEADS = 4
DH = 16
K = 3

NC = 2
NS = 16
NPHASE = 2
RP = 25088
NP = NC * NPHASE * RP
RT = RP // NS
ACC_ROWS = RP + 16
ZR = 112
EC = E // NS
BE = 2000
NBLK = EC // BE
WIN = 128
CAP = BE + WIN + 16


def _segsum_body(src_hbm, dst_hbm, vals_hbm, out_hbm,
                 acc_sh, dstb, srcb, csrc, cdst, crow, rowbuf, zbuf):
    cid = lax.axis_index("c")
    sid = lax.axis_index("s")
    iota = lax.iota(jnp.int32, 16)
    zv = jnp.zeros((16,), jnp.float32)

    def zfill(i, _):
        r = i // 4
        k = i - r * 4
        zbuf[r, pl.ds(k * 16, 16)] = zv
        return 0
    lax.fori_loop(0, ZR * 4, zfill, 0)

    tbase = sid * EC

    for p in range(NPHASE):
        lo = (cid * NPHASE + p) * RP

        def zacc(j, _):
            pltpu.sync_copy(zbuf, acc_sh.at[pl.ds(sid * RT + j * ZR, ZR)])
            return 0
        lax.fori_loop(0, RT // ZR, zacc, 0)
        plsc.subcore_barrier()

        def block(b, _):
            pltpu.sync_copy(dst_hbm.at[pl.ds(tbase + b * BE, BE)], dstb)
            pltpu.sync_copy(src_hbm.at[pl.ds(tbase + b * BE, BE)], srcb)

            def grp(i, cnt):
                d = dstb[pl.ds(i * 16, 16)]
                s = srcb[pl.ds(i * 16, 16)]
                dl = d - lo
                m = (dl >= 0) & (dl < RP)
                mi = m.astype(jnp.int32)
                pos = cnt + plsc.cumsum(mi) - 1
                plsc.store_scatter(csrc, [pos], s, mask=m)
                plsc.store_scatter(cdst, [pos], dl, mask=m)
                return cnt + plsc.all_reduce_population_count(m)

            cntv = lax.fori_loop(0, BE // 16, grp,
                                 jnp.zeros((16,), jnp.int32))
            cnt = lax.reduce_max(cntv, (0,))
            nwin = (cnt + (WIN - 1)) // WIN
            tot = nwin * WIN

            def pad(j, _):
                idx = cnt + j * 16 + iota
                mpad = idx < tot
                plsc.store_scatter(cdst, [idx], RP + iota, mask=mpad)
                plsc.store_scatter(csrc, [idx], iota * 8, mask=mpad)
                return 0
            lax.fori_loop(0, WIN // 16, pad, 0)

            def win(w, _):
                pltpu.sync_copy(vals_hbm.at[csrc.at[pl.ds(w * WIN, WIN)]],
                                rowbuf)
                for k in range(WIN // 16):
                    crow[0, pl.ds(k * 16, 16)] = cdst[pl.ds(w * WIN + k * 16, 16)]
                pltpu.sync_copy(rowbuf, acc_sh.at[crow.at[0]], add=True)
                return 0
            lax.fori_loop(0, nwin, win, 0)
            return 0

        lax.fori_loop(0, NBLK, block, 0)
        plsc.subcore_barrier()
        pltpu.sync_copy(acc_sh.at[pl.ds(sid * RT, RT)],
                        out_hbm.at[pl.ds(lo + sid * RT, RT)])
        plsc.subcore_barrier()


@jax.jit
def _sc_segsum(src, dst, vals):
    mesh = plsc.VectorSubcoreMesh(core_axis_name="c", subcore_axis_name="s",
                                  num_cores=NC, num_subcores=NS)
    f = pl.kernel(
        _segsum_body,
        out_type=jax.ShapeDtypeStruct((NP, H), jnp.float32),
        mesh=mesh,
        compiler_params=pltpu.CompilerParams(needs_layout_passes=False,
                                             use_tc_tiling_on_sc=False),
        scratch_types=[
            pltpu.VMEM_SHARED((ACC_ROWS, H), jnp.float32),
            pltpu.VMEM((BE,), jnp.int32),
            pltpu.VMEM((BE,), jnp.int32),
            pltpu.VMEM((CAP,), jnp.int32),
            pltpu.VMEM((CAP,), jnp.int32),
            pltpu.VMEM((1, WIN), jnp.int32),
            pltpu.VMEM((WIN, H), jnp.float32),
            pltpu.VMEM((ZR, H), jnp.float32),
        ],
    )
    return f(src, dst, vals)


def _gelu_exact(v):
    return 0.5 * v * (1.0 + lax.erf(v * (2.0 ** -0.5)))


def _head_body(hcat_ref, wc1_ref, bc1_ref, wc2_ref, bc2_ref, out_ref):
    h = hcat_ref[...] @ wc1_ref[...] + bc1_ref[...]
    h = _gelu_exact(h)
    logits = h @ wc2_ref[...] + bc2_ref[...]
    out_ref[...] = jax.nn.log_softmax(logits, axis=-1)


def _head(hcat, Wc1, bc1, Wc2, bc2):
    return pl.pallas_call(
        _head_body,
        out_shape=jax.ShapeDtypeStruct((G, 2), jnp.float32),
    )(hcat, Wc1, bc1.reshape(1, H), Wc2, bc2.reshape(1, 2))


def _graph_norm(x, batch, counts, w, b, ms):
    mean = jax.ops.segment_sum(x, batch, G) / counts
    out = x - mean[batch] * ms
    var = jax.ops.segment_sum(out * out, batch, G) / counts
    out = out / jnp.sqrt(var + 1e-5)[batch]
    return out * w + b


def kernel(x, W_enc, b_enc, W_gat, att_src, att_dst, b_gat, gn1_w, gn1_b,
           gn1_ms, W_tag, b_tag, gn2_w, gn2_b, gn2_ms, W1, W2, b_gc, Wc1,
           bc1, Wc2, bc2, edge_index, batch):
    src = edge_index[0]
    dst = edge_index[1]
    x = jax.nn.gelu(x @ W_enc + b_enc, approximate=False)
    loop = jnp.arange(N, dtype=src.dtype)
    s2 = jnp.concatenate([src, loop])
    d2 = jnp.concatenate([dst, loop])
    xw = (x @ W_gat).reshape(N, HEADS, DH)
    a_s = jnp.sum(xw * att_src[None], axis=-1)
    a_d = jnp.sum(xw * att_dst[None], axis=-1)
    e = jax.nn.leaky_relu(a_s[s2] + a_d[d2], 0.2)
    emax = jax.ops.segment_max(e, d2, N)
    emax = jnp.where(jnp.isfinite(emax), emax, 0.0)
    ee = jnp.exp(e - emax[d2])
    denom = jax.ops.segment_sum(ee, d2, N)
    alpha = ee / (denom[d2] + 1e-16)
    gat = jax.ops.segment_sum(xw[s2] * alpha[:, :, None], d2, N).reshape(N, H) + b_gat
    counts = jnp.maximum(jax.ops.segment_sum(jnp.ones((N, 1), jnp.float32), batch, G), 1.0)
    x = jax.nn.relu(_graph_norm(gat, batch, counts, gn1_w, gn1_b, gn1_ms))
    deg = jax.ops.segment_sum(jnp.ones((E,), jnp.float32), dst, N)
    dis = jnp.where(deg > 0, 1.0 / jnp.sqrt(deg), 0.0)
    out = x @ W_tag[0]
    h = x
    for k in range(1, K + 1):
        g = h * dis[:, None]
        h = _sc_segsum(src, dst, g)[:N] * dis[:, None]
        out = out + h @ W_tag[k]
    out = out + b_tag
    x = jax.nn.relu(_graph_norm(out, batch, counts, gn2_w, gn2_b, gn2_ms))
    agg = _sc_segsum(src, dst, x)[:N]
    x = jax.nn.relu(x @ W1 + agg @ W2 + b_gc)
    xm = jax.ops.segment_max(x, batch, G)
    xm = jnp.where(jnp.isfinite(xm), xm, 0.0)
    xM = jax.ops.segment_sum(x, batch, G) / counts
    hcat = jnp.concatenate([xm, xM], axis=1)
    return _head(hcat, Wc1, bc1, Wc2, bc2)


# speedup vs baseline: 33.1085x; 26.3787x over previous
import functools

import jax
import jax.numpy as jnp
from jax import lax
from jax.experimental import pallas as pl
from jax.experimental.pallas import tpu as pltpu
from jax.experimental.pallas import tpu_sc as plsc

N = 100000
E = 3200000
G = 64
F = 8
H = 64
HEADS = 4
DH = 16
K = 3

NC = 2
NS = 16
NPHASE = 2
RP = 25088
NP = NC * NPHASE * RP
RT = RP // NS
ACC_ROWS = RP + 16
ZR = 112
EC = E // NS
BE = 2000
NBLK = EC // BE
WIN = 128
CAP = BE + WIN + 16

PHG = 3
RPG = 18816
NPG = NC * PHG * RPG
RTG = RPG // NS
ZRG = 84
ACC_ROWS_G = RPG + 16


def _segsum_body(src_hbm, dst_hbm, vals_hbm, out_hbm,
                 acc_sh, dstb, srcb, csrc, cdst, crow, rowbuf, zbuf):
    cid = lax.axis_index("c")
    sid = lax.axis_index("s")
    iota = lax.iota(jnp.int32, 16)
    zv = jnp.zeros((16,), jnp.float32)

    def zfill(i, _):
        r = i // 4
        k = i - r * 4
        zbuf[r, pl.ds(k * 16, 16)] = zv
        return 0
    lax.fori_loop(0, ZR * 4, zfill, 0)

    tbase = sid * EC

    for p in range(NPHASE):
        lo = (cid * NPHASE + p) * RP

        def zacc(j, _):
            pltpu.sync_copy(zbuf, acc_sh.at[pl.ds(sid * RT + j * ZR, ZR)])
            return 0
        lax.fori_loop(0, RT // ZR, zacc, 0)
        plsc.subcore_barrier()

        def block(b, _):
            pltpu.sync_copy(dst_hbm.at[pl.ds(tbase + b * BE, BE)], dstb)
            pltpu.sync_copy(src_hbm.at[pl.ds(tbase + b * BE, BE)], srcb)

            def grp(i, cnt):
                d = dstb[pl.ds(i * 16, 16)]
                s = srcb[pl.ds(i * 16, 16)]
                dl = d - lo
                m = (dl >= 0) & (dl < RP)
                mi = m.astype(jnp.int32)
                pos = cnt + plsc.cumsum(mi) - 1
                plsc.store_scatter(csrc, [pos], s, mask=m)
                plsc.store_scatter(cdst, [pos], dl, mask=m)
                return cnt + plsc.all_reduce_population_count(m)

            cntv = lax.fori_loop(0, BE // 16, grp,
                                 jnp.zeros((16,), jnp.int32))
            cnt = lax.reduce_max(cntv, (0,))
            nwin = (cnt + (WIN - 1)) // WIN
            tot = nwin * WIN

            def pad(j, _):
                idx = cnt + j * 16 + iota
                mpad = idx < tot
                plsc.store_scatter(cdst, [idx], RP + iota, mask=mpad)
                plsc.store_scatter(csrc, [idx], iota * 8, mask=mpad)
                return 0
            lax.fori_loop(0, WIN // 16, pad, 0)

            def win(w, _):
                pltpu.sync_copy(vals_hbm.at[csrc.at[pl.ds(w * WIN, WIN)]],
                                rowbuf)
                for k in range(WIN // 16):
                    crow[0, pl.ds(k * 16, 16)] = cdst[pl.ds(w * WIN + k * 16, 16)]
                pltpu.sync_copy(rowbuf, acc_sh.at[crow.at[0]], add=True)
                return 0
            lax.fori_loop(0, nwin, win, 0)
            return 0

        lax.fori_loop(0, NBLK, block, 0)
        plsc.subcore_barrier()
        pltpu.sync_copy(acc_sh.at[pl.ds(sid * RT, RT)],
                        out_hbm.at[pl.ds(lo + sid * RT, RT)])
        plsc.subcore_barrier()


@jax.jit
def _sc_segsum(src, dst, vals):
    mesh = plsc.VectorSubcoreMesh(core_axis_name="c", subcore_axis_name="s",
                                  num_cores=NC, num_subcores=NS)
    f = pl.kernel(
        _segsum_body,
        out_type=jax.ShapeDtypeStruct((NP, H), jnp.float32),
        mesh=mesh,
        compiler_params=pltpu.CompilerParams(needs_layout_passes=False,
                                             use_tc_tiling_on_sc=False),
        scratch_types=[
            pltpu.VMEM_SHARED((ACC_ROWS, H), jnp.float32),
            pltpu.VMEM((BE,), jnp.int32),
            pltpu.VMEM((BE,), jnp.int32),
            pltpu.VMEM((CAP,), jnp.int32),
            pltpu.VMEM((CAP,), jnp.int32),
            pltpu.VMEM((1, WIN), jnp.int32),
            pltpu.VMEM((WIN, H), jnp.float32),
            pltpu.VMEM((ZR, H), jnp.float32),
        ],
    )
    return f(src, dst, vals)


def _gat_body(src_hbm, dst_hbm, xw_hbm, av_hbm, numer_hbm, denom_hbm,
              accn_sh, accd_sh, dstb, srcb, csrc, cdst, cdg, crow,
              rowbuf, avsb, avdb, dnb, zbuf, zbufd):
    cid = lax.axis_index("c")
    sid = lax.axis_index("s")
    iota = lax.iota(jnp.int32, 16)
    sh4 = (iota & 3) + 4
    cvec = jnp.where(iota == 4, 1.0, 0.0).astype(jnp.float32)
    zv = jnp.zeros((16,), jnp.float32)

    def zfill(i, _):
        r = i // 4
        k = i - r * 4
        zbuf[r, pl.ds(k * 16, 16)] = zv
        return 0
    lax.fori_loop(0, ZRG * 4, zfill, 0)

    def zfilld(i, _):
        rows = 2 * i + (iota >= 8).astype(jnp.int32)
        plsc.store_scatter(zbufd, [rows, iota & 7], zv)
        return 0
    lax.fori_loop(0, ZRG // 2, zfilld, 0)

    tbase = sid * EC

    for p in range(PHG):
        lo = (cid * PHG + p) * RPG

        def zacc(j, _):
            pltpu.sync_copy(zbuf, accn_sh.at[pl.ds(sid * RTG + j * ZRG, ZRG)])
            pltpu.sync_copy(zbufd, accd_sh.at[pl.ds(sid * RTG + j * ZRG, ZRG)])
            return 0
        lax.fori_loop(0, RTG // ZRG, zacc, 0)
        plsc.subcore_barrier()

        def block(b, _):
            pltpu.sync_copy(dst_hbm.at[pl.ds(tbase + b * BE, BE)], dstb)
            pltpu.sync_copy(src_hbm.at[pl.ds(tbase + b * BE, BE)], srcb)

            def grp(i, cntv):
                d = dstb[pl.ds(i * 16, 16)]
                s = srcb[pl.ds(i * 16, 16)]
                dl = d - lo
                m = (dl >= 0) & (dl < RPG)
                mi = m.astype(jnp.int32)
                pos = cntv + plsc.cumsum(mi) - 1
                plsc.store_scatter(csrc, [pos], s, mask=m)
                plsc.store_scatter(cdst, [pos], dl, mask=m)
                plsc.store_scatter(cdg, [pos], d, mask=m)
                return cntv + plsc.all_reduce_population_count(m)

            cntv = lax.fori_loop(0, BE // 16, grp,
                                 jnp.zeros((16,), jnp.int32))
            cnt = lax.reduce_max(cntv, (0,))
            nwin = (cnt + (WIN - 1)) // WIN
            tot = nwin * WIN

            def pad(j, _):
                idx = cnt + j * 16 + iota
                mpad = idx < tot
                plsc.store_scatter(cdst, [idx], RPG + iota, mask=mpad)
                plsc.store_scatter(csrc, [idx], iota * 8, mask=mpad)
                plsc.store_scatter(cdg, [idx], iota * 8, mask=mpad)
                return 0
            lax.fori_loop(0, WIN // 16, pad, 0)

            def win(w, _):
                pltpu.sync_copy(xw_hbm.at[csrc.at[pl.ds(w * WIN, WIN)]],
                                rowbuf)
                pltpu.sync_copy(av_hbm.at[csrc.at[pl.ds(w * WIN, WIN)]],
                                avsb)
                pltpu.sync_copy(av_hbm.at[cdg.at[pl.ds(w * WIN, WIN)]],
                                avdb)

                def edge(r, _):
                    rv = jnp.full((16,), r, jnp.int32)
                    va = avsb[r, pl.ds(0, 16)]
                    vd = plsc.load_gather(avdb, [rv, sh4])
                    e4 = va + vd
                    e4 = jnp.where(e4 > 0, e4, 0.2 * e4)
                    ee = jnp.exp(e4)
                    plsc.store_scatter(dnb, [rv, iota],
                                       jnp.where(iota < 4, ee, cvec),
                                       mask=iota < 8)
                    for h in range(HEADS):
                        sc = plsc.load_gather(
                            dnb, [rv, jnp.full((16,), h, jnp.int32)])
                        rowbuf[r, pl.ds(h * 16, 16)] = (
                            rowbuf[r, pl.ds(h * 16, 16)] * sc)
                    return 0
                lax.fori_loop(0, WIN, edge, 0)

                for k in range(WIN // 16):
                    crow[0, pl.ds(k * 16, 16)] = cdst[pl.ds(w * WIN + k * 16, 16)]
                pltpu.sync_copy(rowbuf, accn_sh.at[crow.at[0]], add=True)
                pltpu.sync_copy(dnb, accd_sh.at[crow.at[0]], add=True)
                return 0
            lax.fori_loop(0, nwin, win, 0)
            return 0

        lax.fori_loop(0, NBLK, block, 0)
        plsc.subcore_barrier()
        pltpu.sync_copy(accn_sh.at[pl.ds(sid * RTG, RTG)],
                        numer_hbm.at[pl.ds(lo + sid * RTG, RTG)])
        pltpu.sync_copy(accd_sh.at[pl.ds(sid * RTG, RTG)],
                        denom_hbm.at[pl.ds(lo + sid * RTG, RTG)])
        plsc.subcore_barrier()


@jax.jit
def _sc_gat(src, dst, xw, av):
    mesh = plsc.VectorSubcoreMesh(core_axis_name="c", subcore_axis_name="s",
                                  num_cores=NC, num_subcores=NS)
    f = pl.kernel(
        _gat_body,
        out_type=(jax.ShapeDtypeStruct((NPG, H), jnp.float32),
                  jax.ShapeDtypeStruct((NPG, 8), jnp.float32)),
        mesh=mesh,
        compiler_params=pltpu.CompilerParams(needs_layout_passes=False,
                                             use_tc_tiling_on_sc=False),
        scratch_types=[
            pltpu.VMEM_SHARED((ACC_ROWS_G, H), jnp.float32),
            pltpu.VMEM_SHARED((ACC_ROWS_G, 8), jnp.float32),
            pltpu.VMEM((BE,), jnp.int32),
            pltpu.VMEM((BE,), jnp.int32),
            pltpu.VMEM((CAP,), jnp.int32),
            pltpu.VMEM((CAP,), jnp.int32),
            pltpu.VMEM((CAP,), jnp.int32),
            pltpu.VMEM((1, WIN), jnp.int32),
            pltpu.VMEM((WIN, H), jnp.float32),
            pltpu.VMEM((WIN, 16), jnp.float32),
            pltpu.VMEM((WIN, 16), jnp.float32),
            pltpu.VMEM((WIN, 8), jnp.float32),
            pltpu.VMEM((ZRG, H), jnp.float32),
            pltpu.VMEM((ZRG, 8), jnp.float32),
        ],
    )
    return f(src, dst, xw, av)


def _gelu_exact(v):
    return 0.5 * v * (1.0 + lax.erf(v * (2.0 ** -0.5)))


def _head_body(hcat_ref, wc1_ref, bc1_ref, wc2_ref, bc2_ref, out_ref):
    h = hcat_ref[...] @ wc1_ref[...] + bc1_ref[...]
    h = _gelu_exact(h)
    logits = h @ wc2_ref[...] + bc2_ref[...]
    out_ref[...] = jax.nn.log_softmax(logits, axis=-1)


def _head(hcat, Wc1, bc1, Wc2, bc2):
    return pl.pallas_call(
        _head_body,
        out_shape=jax.ShapeDtypeStruct((G, 2), jnp.float32),
    )(hcat, Wc1, bc1.reshape(1, H), Wc2, bc2.reshape(1, 2))


def _graph_norm(x, batch, counts, w, b, ms):
    mean = jax.ops.segment_sum(x, batch, G) / counts
    out = x - mean[batch] * ms
    var = jax.ops.segment_sum(out * out, batch, G) / counts
    out = out / jnp.sqrt(var + 1e-5)[batch]
    return out * w + b


def kernel(x, W_enc, b_enc, W_gat, att_src, att_dst, b_gat, gn1_w, gn1_b,
           gn1_ms, W_tag, b_tag, gn2_w, gn2_b, gn2_ms, W1, W2, b_gc, Wc1,
           bc1, Wc2, bc2, edge_index, batch):
    src = edge_index[0]
    dst = edge_index[1]
    x = jax.nn.gelu(x @ W_enc + b_enc, approximate=False)
    xw = x @ W_gat
    a_s = (xw.reshape(N, HEADS, DH) * att_src[None]).sum(-1)
    a_d = (xw.reshape(N, HEADS, DH) * att_dst[None]).sum(-1)
    av = jnp.concatenate([a_s, a_d, jnp.zeros((N, 8), jnp.float32)], axis=1)
    numer, den = _sc_gat(src, dst, xw, av)
    e_self = a_s + a_d
    ee_self = jnp.exp(jnp.where(e_self > 0, e_self, 0.2 * e_self))
    num_tot = numer[:N] + jnp.repeat(ee_self, DH, axis=1) * xw
    den_tot = den[:N, :HEADS] + ee_self
    gat = num_tot / (jnp.repeat(den_tot, DH, axis=1) + 1e-16) + b_gat
    counts = jnp.maximum(jax.ops.segment_sum(jnp.ones((N, 1), jnp.float32), batch, G), 1.0)
    x = jax.nn.relu(_graph_norm(gat, batch, counts, gn1_w, gn1_b, gn1_ms))
    deg = den[:N, 4]
    dis = jnp.where(deg > 0, 1.0 / jnp.sqrt(deg), 0.0)
    out = x @ W_tag[0]
    h = x
    for k in range(1, K + 1):
        g = h * dis[:, None]
        h = _sc_segsum(src, dst, g)[:N] * dis[:, None]
        out = out + h @ W_tag[k]
    out = out + b_tag
    x = jax.nn.relu(_graph_norm(out, batch, counts, gn2_w, gn2_b, gn2_ms))
    agg = _sc_segsum(src, dst, x)[:N]
    x = jax.nn.relu(x @ W1 + agg @ W2 + b_gc)
    xm = jax.ops.segment_max(x, batch, G)
    xm = jnp.where(jnp.isfinite(xm), xm, 0.0)
    xM = jax.ops.segment_sum(x, batch, G) / counts
    hcat = jnp.concatenate([xm, xM], axis=1)
    return _head(hcat, Wc1, bc1, Wc2, bc2)


# speedup vs baseline: 35.7219x; 1.0789x over previous
import functools

import jax
import jax.numpy as jnp
from jax import lax
from jax.experimental import pallas as pl
from jax.experimental.pallas import tpu as pltpu
from jax.experimental.pallas import tpu_sc as plsc

N = 100000
E = 3200000
G = 64
F = 8
H = 64
HEADS = 4
DH = 16
K = 3

NC = 2
NS = 16
NPHASE = 2
RP = 25088
NP = NC * NPHASE * RP
RT = RP // NS
ACC_ROWS = RP + 16
ZR = 112
EC = E // NS
BE = 2000
NBLK = EC // BE
WIN = 128
CAP = BE + WIN + 16

PHG = 3
RPG = 18816
NPG = NC * PHG * RPG
RTG = RPG // NS
ZRG = 84
ACC_ROWS_G = RPG + 16


def _segsum_body(src_hbm, dst_hbm, vals_hbm, out_hbm,
                 acc_sh, dstb, srcb, csrc, cdst, crow, rowbuf, zbuf):
    cid = lax.axis_index("c")
    sid = lax.axis_index("s")
    iota = lax.iota(jnp.int32, 16)
    zv = jnp.zeros((16,), jnp.float32)

    def zfill(i, _):
        r = i // 4
        k = i - r * 4
        zbuf[r, pl.ds(k * 16, 16)] = zv
        return 0
    lax.fori_loop(0, ZR * 4, zfill, 0)

    tbase = sid * EC

    for p in range(NPHASE):
        lo = (cid * NPHASE + p) * RP

        def zacc(j, _):
            pltpu.sync_copy(zbuf, acc_sh.at[pl.ds(sid * RT + j * ZR, ZR)])
            return 0
        lax.fori_loop(0, RT // ZR, zacc, 0)
        plsc.subcore_barrier()

        def block(b, _):
            pltpu.sync_copy(dst_hbm.at[pl.ds(tbase + b * BE, BE)], dstb)
            pltpu.sync_copy(src_hbm.at[pl.ds(tbase + b * BE, BE)], srcb)

            def grp(i, cnt):
                d = dstb[pl.ds(i * 16, 16)]
                s = srcb[pl.ds(i * 16, 16)]
                dl = d - lo
                m = (dl >= 0) & (dl < RP)
                mi = m.astype(jnp.int32)
                pos = cnt + plsc.cumsum(mi) - 1
                plsc.store_scatter(csrc, [pos], s, mask=m)
                plsc.store_scatter(cdst, [pos], dl, mask=m)
                return cnt + plsc.all_reduce_population_count(m)

            cntv = lax.fori_loop(0, BE // 16, grp,
                                 jnp.zeros((16,), jnp.int32))
            cnt = lax.reduce_max(cntv, (0,))
            nwin = (cnt + (WIN - 1)) // WIN
            tot = nwin * WIN

            def pad(j, _):
                idx = cnt + j * 16 + iota
                mpad = idx < tot
                plsc.store_scatter(cdst, [idx], RP + iota, mask=mpad)
                plsc.store_scatter(csrc, [idx], iota * 8, mask=mpad)
                return 0
            lax.fori_loop(0, WIN // 16, pad, 0)

            def win(w, _):
                pltpu.sync_copy(vals_hbm.at[csrc.at[pl.ds(w * WIN, WIN)]],
                                rowbuf)
                for k in range(WIN // 16):
                    crow[0, pl.ds(k * 16, 16)] = cdst[pl.ds(w * WIN + k * 16, 16)]
                pltpu.sync_copy(rowbuf, acc_sh.at[crow.at[0]], add=True)
                return 0
            lax.fori_loop(0, nwin, win, 0)
            return 0

        lax.fori_loop(0, NBLK, block, 0)
        plsc.subcore_barrier()
        pltpu.sync_copy(acc_sh.at[pl.ds(sid * RT, RT)],
                        out_hbm.at[pl.ds(lo + sid * RT, RT)])
        plsc.subcore_barrier()


@jax.jit
def _sc_segsum(src, dst, vals):
    mesh = plsc.VectorSubcoreMesh(core_axis_name="c", subcore_axis_name="s",
                                  num_cores=NC, num_subcores=NS)
    f = pl.kernel(
        _segsum_body,
        out_type=jax.ShapeDtypeStruct((NP, H), jnp.float32),
        mesh=mesh,
        compiler_params=pltpu.CompilerParams(needs_layout_passes=False,
                                             use_tc_tiling_on_sc=False),
        scratch_types=[
            pltpu.VMEM_SHARED((ACC_ROWS, H), jnp.float32),
            pltpu.VMEM((BE,), jnp.int32),
            pltpu.VMEM((BE,), jnp.int32),
            pltpu.VMEM((CAP,), jnp.int32),
            pltpu.VMEM((CAP,), jnp.int32),
            pltpu.VMEM((1, WIN), jnp.int32),
            pltpu.VMEM((WIN, H), jnp.float32),
            pltpu.VMEM((ZR, H), jnp.float32),
        ],
    )
    return f(src, dst, vals)


def _gat_body(src_hbm, dst_hbm, xw_hbm, av_hbm, numer_hbm, denom_hbm,
              accn_sh, accd_sh, dstb, srcb, csrc, cdst, cdg, crow,
              rowbuf, avsb, avdb, dnb, zbuf, zbufd, semx, sems, semd):
    cid = lax.axis_index("c")
    sid = lax.axis_index("s")
    iota = lax.iota(jnp.int32, 16)
    sh4 = (iota & 3) + 4
    cvec = jnp.where(iota == 4, 1.0, 0.0).astype(jnp.float32)
    zv = jnp.zeros((16,), jnp.float32)

    def zfill(i, _):
        r = i // 4
        k = i - r * 4
        zbuf[r, pl.ds(k * 16, 16)] = zv
        return 0
    lax.fori_loop(0, ZRG * 4, zfill, 0)

    def zfilld(i, _):
        rows = 2 * i + (iota >= 8).astype(jnp.int32)
        plsc.store_scatter(zbufd, [rows, iota & 7], zv)
        return 0
    lax.fori_loop(0, ZRG // 2, zfilld, 0)

    tbase = sid * EC

    for p in range(PHG):
        lo = (cid * PHG + p) * RPG

        def zacc(j, _):
            pltpu.sync_copy(zbuf, accn_sh.at[pl.ds(sid * RTG + j * ZRG, ZRG)])
            pltpu.sync_copy(zbufd, accd_sh.at[pl.ds(sid * RTG + j * ZRG, ZRG)])
            return 0
        lax.fori_loop(0, RTG // ZRG, zacc, 0)
        plsc.subcore_barrier()

        def block(b, _):
            pltpu.sync_copy(dst_hbm.at[pl.ds(tbase + b * BE, BE)], dstb)
            pltpu.sync_copy(src_hbm.at[pl.ds(tbase + b * BE, BE)], srcb)

            def grp(i, cntv):
                d = dstb[pl.ds(i * 16, 16)]
                s = srcb[pl.ds(i * 16, 16)]
                dl = d - lo
                m = (dl >= 0) & (dl < RPG)
                mi = m.astype(jnp.int32)
                pos = cntv + plsc.cumsum(mi) - 1
                plsc.store_scatter(csrc, [pos], s, mask=m)
                plsc.store_scatter(cdst, [pos], dl, mask=m)
                plsc.store_scatter(cdg, [pos], d, mask=m)
                return cntv + plsc.all_reduce_population_count(m)

            cntv = lax.fori_loop(0, BE // 16, grp,
                                 jnp.zeros((16,), jnp.int32))
            cnt = lax.reduce_max(cntv, (0,))
            nwin = (cnt + (WIN - 1)) // WIN
            tot = nwin * WIN

            def pad(j, _):
                idx = cnt + j * 16 + iota
                mpad = idx < tot
                plsc.store_scatter(cdst, [idx], RPG + iota, mask=mpad)
                plsc.store_scatter(csrc, [idx], iota * 8, mask=mpad)
                plsc.store_scatter(cdg, [idx], iota * 8, mask=mpad)
                return 0
            lax.fori_loop(0, WIN // 16, pad, 0)

            def win(w, _):
                cx = pltpu.async_copy(
                    xw_hbm.at[csrc.at[pl.ds(w * WIN, WIN)]], rowbuf, semx)
                cs = pltpu.async_copy(
                    av_hbm.at[csrc.at[pl.ds(w * WIN, WIN)]], avsb, sems)
                cd = pltpu.async_copy(
                    av_hbm.at[cdg.at[pl.ds(w * WIN, WIN)]], avdb, semd)
                cs.wait()
                cd.wait()

                def eec(r, _):
                    rv = jnp.full((16,), r, jnp.int32)
                    va = avsb[r, pl.ds(0, 16)]
                    vd = plsc.load_gather(avdb, [rv, sh4])
                    e4 = va + vd
                    e4 = jnp.where(e4 > 0, e4, 0.2 * e4)
                    ee = jnp.exp(e4)
                    plsc.store_scatter(dnb, [rv, iota],
                                       jnp.where(iota < 4, ee, cvec),
                                       mask=iota < 8)
                    return 0
                lax.fori_loop(0, WIN, eec, 0)
                cx.wait()

                def edge(r, _):
                    rv = jnp.full((16,), r, jnp.int32)
                    for h in range(HEADS):
                        sc = plsc.load_gather(
                            dnb, [rv, jnp.full((16,), h, jnp.int32)])
                        rowbuf[r, pl.ds(h * 16, 16)] = (
                            rowbuf[r, pl.ds(h * 16, 16)] * sc)
                    return 0
                lax.fori_loop(0, WIN, edge, 0)

                for k in range(WIN // 16):
                    crow[0, pl.ds(k * 16, 16)] = cdst[pl.ds(w * WIN + k * 16, 16)]
                pltpu.sync_copy(rowbuf, accn_sh.at[crow.at[0]], add=True)
                pltpu.sync_copy(dnb, accd_sh.at[crow.at[0]], add=True)
                return 0
            lax.fori_loop(0, nwin, win, 0)
            return 0

        lax.fori_loop(0, NBLK, block, 0)
        plsc.subcore_barrier()
        pltpu.sync_copy(accn_sh.at[pl.ds(sid * RTG, RTG)],
                        numer_hbm.at[pl.ds(lo + sid * RTG, RTG)])
        pltpu.sync_copy(accd_sh.at[pl.ds(sid * RTG, RTG)],
                        denom_hbm.at[pl.ds(lo + sid * RTG, RTG)])
        plsc.subcore_barrier()


@jax.jit
def _sc_gat(src, dst, xw, av):
    mesh = plsc.VectorSubcoreMesh(core_axis_name="c", subcore_axis_name="s",
                                  num_cores=NC, num_subcores=NS)
    f = pl.kernel(
        _gat_body,
        out_type=(jax.ShapeDtypeStruct((NPG, H), jnp.float32),
                  jax.ShapeDtypeStruct((NPG, 8), jnp.float32)),
        mesh=mesh,
        compiler_params=pltpu.CompilerParams(needs_layout_passes=False,
                                             use_tc_tiling_on_sc=False),
        scratch_types=[
            pltpu.VMEM_SHARED((ACC_ROWS_G, H), jnp.float32),
            pltpu.VMEM_SHARED((ACC_ROWS_G, 8), jnp.float32),
            pltpu.VMEM((BE,), jnp.int32),
            pltpu.VMEM((BE,), jnp.int32),
            pltpu.VMEM((CAP,), jnp.int32),
            pltpu.VMEM((CAP,), jnp.int32),
            pltpu.VMEM((CAP,), jnp.int32),
            pltpu.VMEM((1, WIN), jnp.int32),
            pltpu.VMEM((WIN, H), jnp.float32),
            pltpu.VMEM((WIN, 16), jnp.float32),
            pltpu.VMEM((WIN, 16), jnp.float32),
            pltpu.VMEM((WIN, 8), jnp.float32),
            pltpu.VMEM((ZRG, H), jnp.float32),
            pltpu.VMEM((ZRG, 8), jnp.float32),
            pltpu.SemaphoreType.DMA,
            pltpu.SemaphoreType.DMA,
            pltpu.SemaphoreType.DMA,
        ],
    )
    return f(src, dst, xw, av)


def _gelu_exact(v):
    return 0.5 * v * (1.0 + lax.erf(v * (2.0 ** -0.5)))


def _head_body(hcat_ref, wc1_ref, bc1_ref, wc2_ref, bc2_ref, out_ref):
    h = hcat_ref[...] @ wc1_ref[...] + bc1_ref[...]
    h = _gelu_exact(h)
    logits = h @ wc2_ref[...] + bc2_ref[...]
    out_ref[...] = jax.nn.log_softmax(logits, axis=-1)


def _head(hcat, Wc1, bc1, Wc2, bc2):
    return pl.pallas_call(
        _head_body,
        out_shape=jax.ShapeDtypeStruct((G, 2), jnp.float32),
    )(hcat, Wc1, bc1.reshape(1, H), Wc2, bc2.reshape(1, 2))


def _graph_norm(x, batch, counts, w, b, ms):
    mean = jax.ops.segment_sum(x, batch, G) / counts
    out = x - mean[batch] * ms
    var = jax.ops.segment_sum(out * out, batch, G) / counts
    out = out / jnp.sqrt(var + 1e-5)[batch]
    return out * w + b


def kernel(x, W_enc, b_enc, W_gat, att_src, att_dst, b_gat, gn1_w, gn1_b,
           gn1_ms, W_tag, b_tag, gn2_w, gn2_b, gn2_ms, W1, W2, b_gc, Wc1,
           bc1, Wc2, bc2, edge_index, batch):
    src = edge_index[0]
    dst = edge_index[1]
    x = jax.nn.gelu(x @ W_enc + b_enc, approximate=False)
    xw = x @ W_gat
    a_s = (xw.reshape(N, HEADS, DH) * att_src[None]).sum(-1)
    a_d = (xw.reshape(N, HEADS, DH) * att_dst[None]).sum(-1)
    av = jnp.concatenate([a_s, a_d, jnp.zeros((N, 8), jnp.float32)], axis=1)
    numer, den = _sc_gat(src, dst, xw, av)
    e_self = a_s + a_d
    ee_self = jnp.exp(jnp.where(e_self > 0, e_self, 0.2 * e_self))
    num_tot = numer[:N] + jnp.repeat(ee_self, DH, axis=1) * xw
    den_tot = den[:N, :HEADS] + ee_self
    gat = num_tot / (jnp.repeat(den_tot, DH, axis=1) + 1e-16) + b_gat
    counts = jnp.maximum(jax.ops.segment_sum(jnp.ones((N, 1), jnp.float32), batch, G), 1.0)
    x = jax.nn.relu(_graph_norm(gat, batch, counts, gn1_w, gn1_b, gn1_ms))
    deg = den[:N, 4]
    dis = jnp.where(deg > 0, 1.0 / jnp.sqrt(deg), 0.0)
    out = x @ W_tag[0]
    h = x
    for k in range(1, K + 1):
        g = h * dis[:, None]
        h = _sc_segsum(src, dst, g)[:N] * dis[:, None]
        out = out + h @ W_tag[k]
    out = out + b_tag
    x = jax.nn.relu(_graph_norm(out, batch, counts, gn2_w, gn2_b, gn2_ms))
    agg = _sc_segsum(src, dst, x)[:N]
    x = jax.nn.relu(x @ W1 + agg @ W2 + b_gc)
    xm = jax.ops.segment_max(x, batch, G)
    xm = jnp.where(jnp.isfinite(xm), xm, 0.0)
    xM = jax.ops.segment_sum(x, batch, G) / counts
    hcat = jnp.concatenate([xm, xM], axis=1)
    return _head(hcat, Wc1, bc1, Wc2, bc2)
